# baseline (speedup 1.0000x reference)
import jax
import jax.numpy as jnp
from jax import lax
from jax.experimental import pallas as pl
from jax.experimental.pallas import tpu as pltpu
from jax.experimental.pallas import tpu_sc as plsc

SIZE = 1000000
DIM = 64
BATCH = 16384
BAG = 50

NCORE = 2
NSUB = 16
NW = NCORE * NSUB
LANES = 16
DV = DIM // LANES

TBLK = 128
NBLK = SIZE // TBLK
TAIL = SIZE - NBLK * TBLK
OUTW = TBLK * DIM
SLOTS = -(-NBLK // NW)

BPT = BATCH // NW
CB = 8
NCH = BPT // CB
SUB = 4
IPS = CB * BAG // SUB


def _tr_body(wt_hbm, lin_hbm, in_v0, in_v1, in_t, out_v0, out_v1,
             isem0, isem1, osem0, osem1):
    w = lax.axis_index("s") * NCORE + lax.axis_index("c")
    ins = (in_v0, in_v1)
    outs = (out_v0, out_v1)
    isems = (isem0, isem1)
    osems = (osem0, osem1)
    viota64 = lax.iota(jnp.int32, 16) * DIM

    def blk_of(s):
        return w + NW * s

    def issue_in(s, b):
        pltpu.async_copy(wt_hbm.at[:, pl.ds(blk_of(s) * TBLK, TBLK)],
                         ins[b], isems[b])

    def wait_in(s, b):
        pltpu.make_async_copy(wt_hbm.at[:, pl.ds(blk_of(s) * TBLK, TBLK)],
                              ins[b], isems[b]).wait()

    def issue_out(s, b):
        pltpu.async_copy(outs[b],
                         lin_hbm.at[pl.ds(blk_of(s) * OUTW, OUTW)], osems[b])

    def wait_out(s, b):
        pltpu.make_async_copy(outs[b],
                              lin_hbm.at[pl.ds(blk_of(s) * OUTW, OUTW)],
                              osems[b]).wait()

    def transpose_block(b):
        iv = ins[b]
        ov = outs[b]

        def dline(d, _):
            for i0 in range(0, TBLK, LANES):
                val = iv[d, pl.ds(i0, LANES)]
                plsc.store_scatter(ov, [viota64 + (d + i0 * DIM)], val)
            return 0

        lax.fori_loop(0, DIM, dline, 0)

    def slot(s, b, may_have_pending_out):
        valid = blk_of(s) < NBLK

        @pl.when(jnp.logical_and(s + 1 < SLOTS, blk_of(s + 1) < NBLK))
        def _():
            issue_in(s + 1, 1 - b)

        @pl.when(valid)
        def _():
            wait_in(s, b)

        if may_have_pending_out:
            @pl.when(jnp.logical_and(s >= 2, blk_of(s - 2) < NBLK))
            def _():
                wait_out(s - 2, b)

        @pl.when(valid)
        def _():
            transpose_block(b)
            issue_out(s, b)

    issue_in(0, 0)

    def pairbody(kk, _):
        slot(2 * kk, 0, True)
        slot(2 * kk + 1, 1, True)
        return 0

    lax.fori_loop(0, SLOTS // 2, pairbody, 0)
    if SLOTS % 2:
        slot(SLOTS - 1, 0, True)

    for s in (SLOTS - 2, SLOTS - 1):
        @pl.when(blk_of(s) < NBLK)
        def _(s=s):
            wait_out(s, s % 2)

    @pl.when(w == NW - 1)
    def _():
        pltpu.sync_copy(wt_hbm.at[:, pl.ds(NBLK * TBLK, TAIL)], in_t)

        def dline(d, _):
            for i0 in range(0, TAIL, LANES):
                val = in_t[d, pl.ds(i0, LANES)]
                plsc.store_scatter(out_v0, [viota64 + (d + i0 * DIM)], val)
            return 0

        lax.fori_loop(0, DIM, dline, 0)
        pltpu.sync_copy(out_v0.at[pl.ds(0, TAIL * DIM)],
                        lin_hbm.at[pl.ds(NBLK * OUTW, TAIL * DIM)])


def _bag_body(idx_hbm, scr_hbm, tbl_hbm, out_hbm, idx_v, scr_v, rows_v, out_v,
              sem0, sem1):
    wid = lax.axis_index("s") * NCORE + lax.axis_index("c")
    sems = (sem0, sem1)

    pltpu.sync_copy(idx_hbm.at[wid], idx_v)
    pltpu.sync_copy(scr_hbm.at[wid], scr_v.at[pl.ds(0, BPT * BAG)])

    def issue(g, b):
        for s in range(SUB):
            pltpu.async_copy(
                tbl_hbm.at[idx_v.at[g * SUB + s]],
                rows_v.at[b, pl.ds(s * IPS, IPS)],
                sems[b],
            )

    def drain(g, b):
        for s in range(SUB):
            pltpu.make_async_copy(
                tbl_hbm.at[idx_v.at[g * SUB + s]],
                rows_v.at[b, pl.ds(s * IPS, IPS)],
                sems[b],
            ).wait()

    def compute(g, b):
        sbase = g * (CB * BAG)

        def bag(c, _):
            accs = [jnp.zeros((LANES,), jnp.float32) for _ in range(DV)]
            base = sbase + c * BAG
            for jj in range(0, BAG, LANES):
                svec = scr_v[pl.ds(base + jj, LANES)]
                for lane in range(min(LANES, BAG - jj)):
                    j = jj + lane
                    sc = svec[lane]
                    r = c * BAG + j
                    for t in range(DV):
                        accs[t] = accs[t] + sc * rows_v[b, r,
                                                        pl.ds(t * LANES, LANES)]
            for t in range(DV):
                out_v[c, pl.ds(t * LANES, LANES)] = accs[t]
            return 0

        lax.fori_loop(0, CB, bag, 0)
        pltpu.sync_copy(out_v, out_hbm.at[pl.ds(wid * BPT + g * CB, CB)])

    issue(0, 0)

    def pair(gg, _):
        g0 = 2 * gg
        g1 = g0 + 1
        issue(g1, 1)
        drain(g0, 0)
        compute(g0, 0)

        @pl.when(g1 + 1 < NCH)
        def _():
            issue(g1 + 1, 0)

        drain(g1, 1)
        compute(g1, 1)
        return 0

    lax.fori_loop(0, NCH // 2, pair, 0)


@jax.jit
def _bag_call(idx3, scr2, weight):
    mesh = plsc.VectorSubcoreMesh(core_axis_name="c", subcore_axis_name="s")
    lin = pl.kernel(
        _tr_body,
        out_type=jax.ShapeDtypeStruct((SIZE * DIM,), jnp.float32),
        mesh=mesh,
        scratch_types=[
            pltpu.VMEM((DIM, TBLK), jnp.float32),
            pltpu.VMEM((DIM, TBLK), jnp.float32),
            pltpu.VMEM((DIM, TAIL), jnp.float32),
            pltpu.VMEM((OUTW,), jnp.float32),
            pltpu.VMEM((OUTW,), jnp.float32),
            pltpu.SemaphoreType.DMA,
            pltpu.SemaphoreType.DMA,
            pltpu.SemaphoreType.DMA,
            pltpu.SemaphoreType.DMA,
        ],
        compiler_params=pltpu.CompilerParams(use_tc_tiling_on_sc=True,
                                             needs_layout_passes=False),
    )(weight.T)
    tbl = lin.reshape(SIZE, DIM)
    return pl.kernel(
        _bag_body,
        out_type=jax.ShapeDtypeStruct((BATCH, DIM), jnp.float32),
        mesh=mesh,
        scratch_types=[
            pltpu.VMEM((NCH * SUB, IPS), jnp.int32),
            pltpu.VMEM((BPT * BAG + LANES,), jnp.float32),
            pltpu.VMEM((2, CB * BAG, DIM), jnp.float32),
            pltpu.VMEM((CB, DIM), jnp.float32),
            pltpu.SemaphoreType.DMA,
            pltpu.SemaphoreType.DMA,
        ],
        compiler_params=pltpu.CompilerParams(use_tc_tiling_on_sc=False),
    )(idx3, scr2, tbl)


def kernel(indices, scores, weight):
    idx3 = indices.astype(jnp.int32).reshape(NW, NCH * SUB, IPS)
    scr2 = scores.reshape(NW, BPT * BAG)
    return _bag_call(idx3, scr2, weight)


# speedup vs baseline: 2.1030x; 2.1030x over previous
import jax
import jax.numpy as jnp
from jax import lax
from jax.experimental import pallas as pl
from jax.experimental.pallas import tpu as pltpu
from jax.experimental.pallas import tpu_sc as plsc

SIZE = 1000000
DIM = 64
BATCH = 16384
BAG = 50

NCORE = 2
NSUB = 16
NW = NCORE * NSUB
LANES = 16
DV = DIM // LANES

TBLK = 128
NBLK = SIZE // TBLK
TAIL = SIZE - NBLK * TBLK
OUTW = TBLK * DIM
SLOTS = -(-NBLK // NW)

BPT = BATCH // NW
CB = 8
NCH = BPT // CB
SUB = 4
IPS = CB * BAG // SUB


def _tr_body(wt_hbm, lin_hbm, in_v0, in_v1, in_t, out_v0, out_v1,
             isem0, isem1, osem0, osem1):
    w = lax.axis_index("s") * NCORE + lax.axis_index("c")
    ins = (in_v0, in_v1)
    outs = (out_v0, out_v1)
    isems = (isem0, isem1)
    osems = (osem0, osem1)
    viota = lax.iota(jnp.int32, 16)

    def blk_of(s):
        return w + NW * s

    def issue_in(s, b):
        pltpu.async_copy(wt_hbm.at[:, pl.ds(blk_of(s) * TBLK, TBLK)],
                         ins[b], isems[b])

    def wait_in(s, b):
        pltpu.make_async_copy(wt_hbm.at[:, pl.ds(blk_of(s) * TBLK, TBLK)],
                              ins[b], isems[b]).wait()

    def issue_out(s, b):
        pltpu.async_copy(outs[b],
                         lin_hbm.at[pl.ds(blk_of(s) * OUTW, OUTW)], osems[b])

    def wait_out(s, b):
        pltpu.make_async_copy(outs[b],
                              lin_hbm.at[pl.ds(blk_of(s) * OUTW, OUTW)],
                              osems[b]).wait()

    def transpose_block(b):
        iv = ins[b]
        ov = outs[b]
        for d0 in range(0, DIM, LANES):
            vrow = viota + d0

            def ibody(i0, _, vrow=vrow):
                vi = lax.bitwise_and(viota + i0, TBLK - 1)
                val = plsc.load_gather(iv, [vrow, vi])
                plsc.store_scatter(ov, [lax.shift_left(vi, 6) + vrow], val)
                return 0

            lax.fori_loop(0, TBLK, ibody, 0, unroll=8)

    def slot(s, b, may_have_pending_out):
        valid = blk_of(s) < NBLK

        @pl.when(jnp.logical_and(s + 1 < SLOTS, blk_of(s + 1) < NBLK))
        def _():
            issue_in(s + 1, 1 - b)

        @pl.when(valid)
        def _():
            wait_in(s, b)

        if may_have_pending_out:
            @pl.when(jnp.logical_and(s >= 2, blk_of(s - 2) < NBLK))
            def _():
                wait_out(s - 2, b)

        @pl.when(valid)
        def _():
            transpose_block(b)
            issue_out(s, b)

    issue_in(0, 0)

    def pairbody(kk, _):
        slot(2 * kk, 0, True)
        slot(2 * kk + 1, 1, True)
        return 0

    lax.fori_loop(0, SLOTS // 2, pairbody, 0)
    if SLOTS % 2:
        slot(SLOTS - 1, 0, True)

    for s in (SLOTS - 2, SLOTS - 1):
        @pl.when(blk_of(s) < NBLK)
        def _(s=s):
            wait_out(s, s % 2)

    @pl.when(w == NW - 1)
    def _():
        pltpu.sync_copy(wt_hbm.at[:, pl.ds(NBLK * TBLK, TAIL)], in_t)

        for d0 in range(0, DIM, LANES):
            vrow = viota + d0

            def tbody(i0, _, vrow=vrow):
                vi = lax.bitwise_and(viota + i0, TAIL - 1)
                val = plsc.load_gather(in_t, [vrow, vi])
                plsc.store_scatter(out_v0, [lax.shift_left(vi, 6) + vrow], val)
                return 0

            lax.fori_loop(0, TAIL, tbody, 0, unroll=8)
        pltpu.sync_copy(out_v0.at[pl.ds(0, TAIL * DIM)],
                        lin_hbm.at[pl.ds(NBLK * OUTW, TAIL * DIM)])


def _bag_body(idx_hbm, scr_hbm, tbl_hbm, out_hbm, idx_v, scr_v, rows_v, out_v,
              sem0, sem1):
    wid = lax.axis_index("s") * NCORE + lax.axis_index("c")
    sems = (sem0, sem1)

    pltpu.sync_copy(idx_hbm.at[wid], idx_v)
    pltpu.sync_copy(scr_hbm.at[wid], scr_v.at[pl.ds(0, BPT * BAG)])

    def issue(g, b):
        for s in range(SUB):
            pltpu.async_copy(
                tbl_hbm.at[idx_v.at[g * SUB + s]],
                rows_v.at[b, pl.ds(s * IPS, IPS)],
                sems[b],
            )

    def drain(g, b):
        for s in range(SUB):
            pltpu.make_async_copy(
                tbl_hbm.at[idx_v.at[g * SUB + s]],
                rows_v.at[b, pl.ds(s * IPS, IPS)],
                sems[b],
            ).wait()

    def compute(g, b):
        sbase = g * (CB * BAG)

        def bag(c, _):
            accs = [jnp.zeros((LANES,), jnp.float32) for _ in range(DV)]
            base = sbase + c * BAG
            for jj in range(0, BAG, LANES):
                svec = scr_v[pl.ds(base + jj, LANES)]
                for lane in range(min(LANES, BAG - jj)):
                    j = jj + lane
                    sc = svec[lane]
                    r = c * BAG + j
                    for t in range(DV):
                        accs[t] = accs[t] + sc * rows_v[b, r,
                                                        pl.ds(t * LANES, LANES)]
            for t in range(DV):
                out_v[c, pl.ds(t * LANES, LANES)] = accs[t]
            return 0

        lax.fori_loop(0, CB, bag, 0)
        pltpu.sync_copy(out_v, out_hbm.at[pl.ds(wid * BPT + g * CB, CB)])

    issue(0, 0)

    def pair(gg, _):
        g0 = 2 * gg
        g1 = g0 + 1
        issue(g1, 1)
        drain(g0, 0)
        compute(g0, 0)

        @pl.when(g1 + 1 < NCH)
        def _():
            issue(g1 + 1, 0)

        drain(g1, 1)
        compute(g1, 1)
        return 0

    lax.fori_loop(0, NCH // 2, pair, 0)


@jax.jit
def _bag_call(idx3, scr2, weight):
    mesh = plsc.VectorSubcoreMesh(core_axis_name="c", subcore_axis_name="s")
    lin = pl.kernel(
        _tr_body,
        out_type=jax.ShapeDtypeStruct((SIZE * DIM,), jnp.float32),
        mesh=mesh,
        scratch_types=[
            pltpu.VMEM((DIM, TBLK), jnp.float32),
            pltpu.VMEM((DIM, TBLK), jnp.float32),
            pltpu.VMEM((DIM, TAIL), jnp.float32),
            pltpu.VMEM((OUTW,), jnp.float32),
            pltpu.VMEM((OUTW,), jnp.float32),
            pltpu.SemaphoreType.DMA,
            pltpu.SemaphoreType.DMA,
            pltpu.SemaphoreType.DMA,
            pltpu.SemaphoreType.DMA,
        ],
        compiler_params=pltpu.CompilerParams(use_tc_tiling_on_sc=True,
                                             needs_layout_passes=False),
    )(weight.T)
    tbl = lin.reshape(SIZE, DIM)
    return pl.kernel(
        _bag_body,
        out_type=jax.ShapeDtypeStruct((BATCH, DIM), jnp.float32),
        mesh=mesh,
        scratch_types=[
            pltpu.VMEM((NCH * SUB, IPS), jnp.int32),
            pltpu.VMEM((BPT * BAG + LANES,), jnp.float32),
            pltpu.VMEM((2, CB * BAG, DIM), jnp.float32),
            pltpu.VMEM((CB, DIM), jnp.float32),
            pltpu.SemaphoreType.DMA,
            pltpu.SemaphoreType.DMA,
        ],
        compiler_params=pltpu.CompilerParams(use_tc_tiling_on_sc=False),
    )(idx3, scr2, tbl)


def kernel(indices, scores, weight):
    idx3 = indices.astype(jnp.int32).reshape(NW, NCH * SUB, IPS)
    scr2 = scores.reshape(NW, BPT * BAG)
    return _bag_call(idx3, scr2, weight)
